# baseline (speedup 1.0000x reference)
import functools
import jax
import jax.numpy as jnp
from jax import lax
from jax.experimental import pallas as pl
from jax.experimental.pallas import tpu as pltpu
from jax.experimental.pallas import tpu_sc as plsc

_MESH = plsc.VectorSubcoreMesh(core_axis_name="c", subcore_axis_name="s")


@functools.partial(
    pl.kernel,
    out_type=jax.ShapeDtypeStruct((1, 128), jnp.float32),
    mesh=_MESH,
    scratch_types=[
        pltpu.VMEM((1,), jnp.int32),
        pltpu.VMEM((1, 128), jnp.float32),
        pltpu.SemaphoreType.DMA,
    ],
)
def _sc_gather(idx_hbm, table_hbm, out_hbm, idx_v, rows_v, sem):
    wid = lax.axis_index("s") * 2 + lax.axis_index("c")

    @pl.when(wid == 0)
    def _():
        pltpu.sync_copy(idx_hbm, idx_v)
        pltpu.async_copy(table_hbm.at[idx_v], rows_v, sem).wait()
        pltpu.sync_copy(rows_v, out_hbm)


def _tc_body(row_ref, tag_ref, w1_ref, b1_ref, w2_ref, b2_ref, out_ref):
    row = row_ref[0:1]
    tag = tag_ref[...]
    cat = jnp.concatenate([row, tag], axis=1)
    z1 = lax.dot_general(
        cat, w1_ref[...], (((1,), (1,)), ((), ())),
        preferred_element_type=jnp.float32,
    ) + b1_ref[...]
    a1 = jnp.maximum(z1, 0.0)
    z2 = lax.dot_general(
        a1, w2_ref[...], (((1,), (1,)), ((), ())),
        preferred_element_type=jnp.float32,
    ) + b2_ref[...]
    m = jnp.max(z2, axis=1, keepdims=True)
    s = jnp.sum(jnp.exp(z2 - m), axis=1, keepdims=True)
    out_ref[...] = z2 - m - jnp.log(s)


@jax.jit
def kernel(word_embed_idx, pre_tag_embed, table, W1, b1, W2, b2):
    row = _sc_gather(word_embed_idx.astype(jnp.int32), table)
    return pl.pallas_call(
        _tc_body,
        out_shape=jax.ShapeDtypeStruct((1, 64), jnp.float32),
    )(row, pre_tag_embed, W1, b1.reshape(1, -1), W2, b2.reshape(1, -1))


# speedup vs baseline: 4.9588x; 4.9588x over previous
import jax
import jax.numpy as jnp
from jax import lax
from jax.experimental import pallas as pl
from jax.experimental.pallas import tpu as pltpu


def _body(idx_ref, row_ref, tag_ref, w1_ref, b1_ref, w2_ref, b2_ref, out_ref):
    del idx_ref
    row = row_ref[0]
    tag = tag_ref[...]
    cat = jnp.concatenate([row, tag], axis=1)
    z1 = lax.dot_general(
        cat, w1_ref[...], (((1,), (1,)), ((), ())),
        preferred_element_type=jnp.float32,
    ) + b1_ref[...]
    a1 = jnp.maximum(z1, 0.0)
    z2 = lax.dot_general(
        a1, w2_ref[...], (((1,), (1,)), ((), ())),
        preferred_element_type=jnp.float32,
    ) + b2_ref[...]
    m = jnp.max(z2, axis=1, keepdims=True)
    s = jnp.sum(jnp.exp(z2 - m), axis=1, keepdims=True)
    out_ref[...] = z2 - m - jnp.log(s)


@jax.jit
def kernel(word_embed_idx, pre_tag_embed, table, W1, b1, W2, b2):
    idx = word_embed_idx.astype(jnp.int32)
    grid_spec = pltpu.PrefetchScalarGridSpec(
        num_scalar_prefetch=1,
        grid=(1,),
        in_specs=[
            pl.BlockSpec((1, 1, 128), lambda i, idx_ref: (idx_ref[0], 0, 0)),
            pl.BlockSpec((1, 64), lambda i, idx_ref: (0, 0)),
            pl.BlockSpec((256, 192), lambda i, idx_ref: (0, 0)),
            pl.BlockSpec((1, 256), lambda i, idx_ref: (0, 0)),
            pl.BlockSpec((64, 256), lambda i, idx_ref: (0, 0)),
            pl.BlockSpec((1, 64), lambda i, idx_ref: (0, 0)),
        ],
        out_specs=pl.BlockSpec((1, 64), lambda i, idx_ref: (0, 0)),
    )
    return pl.pallas_call(
        _body,
        grid_spec=grid_spec,
        out_shape=jax.ShapeDtypeStruct((1, 64), jnp.float32),
    )(idx, table.reshape(-1, 1, 128), pre_tag_embed, W1,
      b1.reshape(1, -1), W2, b2.reshape(1, -1))
